# baseline (speedup 1.0000x reference)
import functools

import jax
import jax.numpy as jnp
from jax.experimental import pallas as pl

_EPS = 1e-5
_BM = 3072


def _fused_body(has_stats, has_latb, want_stats, want_y, pool, spb,
                softmax, *refs):
    it = iter(refs)
    x_ref = next(it)
    w_ref = next(it)
    b_ref = next(it)
    mu_ref = next(it) if has_stats else None
    isig_ref = next(it) if has_stats else None
    latb_ref = next(it) if has_latb else None
    y_ref = next(it) if want_y else None
    sum_ref = next(it) if want_stats else None
    ssq_ref = next(it) if want_stats else None
    ymax_ref = next(it) if pool else None

    i = pl.program_id(0)
    a = x_ref[...]
    if has_stats:
        a = jnp.maximum((a - mu_ref[...]) * isig_ref[...], 0.0)
    y = jax.lax.dot_general(a, w_ref[...], (((1,), (1,)), ((), ())),
                            preferred_element_type=jnp.float32)
    y = y + b_ref[...]
    if has_latb:
        bm, o = y.shape
        y = (y.reshape(spb, bm // spb, o) + latb_ref[...]).reshape(bm, o)
    if softmax:
        m = jnp.max(y, axis=1, keepdims=True)
        y = y - (m + jnp.log(jnp.sum(jnp.exp(y - m), axis=1, keepdims=True)))
    if want_y:
        y_ref[...] = y
    if want_stats:
        s = jnp.sum(y, axis=0, keepdims=True)
        ss = jnp.sum(y * y, axis=0, keepdims=True)

        @pl.when(i == 0)
        def _():
            sum_ref[...] = s
            ssq_ref[...] = ss

        @pl.when(i > 0)
        def _():
            sum_ref[...] += s
            ssq_ref[...] += ss
    if pool:
        bm, o = y.shape
        ymax_ref[...] = jnp.max(y.reshape(spb, bm // spb, o), axis=1)[:, None]


def _fused(x, w, b, stats=None, latb=None, pool=False, n_per_sample=None,
           bm=_BM, want_stats=True, want_y=True, softmax=False):
    M, K = x.shape
    O = w.shape[0]
    nt = M // bm
    spb = (bm // n_per_sample) if n_per_sample else 1
    nsamp = (M // n_per_sample) if n_per_sample else 1

    in_specs = [
        pl.BlockSpec((bm, K), lambda i: (i, 0)),
        pl.BlockSpec((O, K), lambda i: (0, 0)),
        pl.BlockSpec((1, O), lambda i: (0, 0)),
    ]
    args = [x, w, b.reshape(1, O)]
    if stats is not None:
        in_specs += [pl.BlockSpec((1, K), lambda i: (0, 0)),
                     pl.BlockSpec((1, K), lambda i: (0, 0))]
        args += [stats[0], stats[1]]
    if latb is not None:
        in_specs += [pl.BlockSpec((spb, 1, O), lambda i: (i, 0, 0))]
        args += [latb.reshape(nsamp, 1, O)]

    out_shape, out_specs = [], []
    if want_y:
        out_shape += [jax.ShapeDtypeStruct((M, O), jnp.float32)]
        out_specs += [pl.BlockSpec((bm, O), lambda i: (i, 0))]
    if want_stats:
        out_shape += [jax.ShapeDtypeStruct((1, O), jnp.float32)] * 2
        out_specs += [pl.BlockSpec((1, O), lambda i: (0, 0))] * 2
    if pool:
        out_shape += [jax.ShapeDtypeStruct((nsamp, 1, O), jnp.float32)]
        out_specs += [pl.BlockSpec((spb, 1, O), lambda i: (i, 0, 0))]

    body = functools.partial(_fused_body, stats is not None,
                             latb is not None, want_stats, want_y, pool, spb,
                             softmax)
    outs = pl.pallas_call(
        body, grid=(nt,), in_specs=in_specs, out_specs=out_specs,
        out_shape=out_shape)(*args)
    outs = list(outs)
    if pool:
        outs[-1] = outs[-1].reshape(nsamp, O)
    return outs


def _gcat_body(*refs):
    (m6, mu6, is6, m5, mu5, is5, m4, mu4, is4, m3, mu3, is3, out) = refs
    out[:, 0:1024] = jnp.maximum((m6[...] - mu6[...]) * is6[...], 0.0)
    out[:, 1024:1536] = jnp.maximum((m5[...] - mu5[...]) * is5[...], 0.0)
    out[:, 1536:1792] = jnp.maximum((m4[...] - mu4[...]) * is4[...], 0.0)
    out[:, 1792:1920] = jnp.maximum((m3[...] - mu3[...]) * is3[...], 0.0)


def _gcat(maxes, stats):
    nsamp = maxes[0].shape[0]
    args, in_specs = [], []
    for m, st in zip(maxes, stats):
        args += [m, st[0], st[1]]
        in_specs += [pl.BlockSpec(m.shape, lambda: (0, 0)),
                     pl.BlockSpec(st[0].shape, lambda: (0, 0)),
                     pl.BlockSpec(st[1].shape, lambda: (0, 0))]
    return pl.pallas_call(
        _gcat_body,
        in_specs=in_specs,
        out_specs=pl.BlockSpec((nsamp, 1920), lambda: (0, 0)),
        out_shape=jax.ShapeDtypeStruct((nsamp, 1920), jnp.float32),
    )(*args)


def _wide_body(relu, precision, x_ref, w_ref, b_ref, y_ref):
    y = jax.lax.dot_general(x_ref[...], w_ref[...], (((1,), (1,)), ((), ())),
                            preferred_element_type=jnp.float32,
                            precision=precision) + b_ref[...]
    if relu:
        y = jnp.maximum(y, 0.0)
    y_ref[...] = y


def _wide(x, w, b, bo=2048, relu=True):
    M, K = x.shape
    O = w.shape[0]
    nt = O // bo
    return pl.pallas_call(
        functools.partial(_wide_body, relu, None),
        grid=(nt,),
        in_specs=[pl.BlockSpec((M, K), lambda i: (0, 0)),
                  pl.BlockSpec((bo, K), lambda i: (i, 0)),
                  pl.BlockSpec((1, bo), lambda i: (0, i))],
        out_specs=pl.BlockSpec((M, bo), lambda i: (0, i)),
        out_shape=jax.ShapeDtypeStruct((M, O), jnp.float32),
    )(x, w, b.reshape(1, O))


def _dsmall(x, w, b, relu=False):
    M, K = x.shape
    O = w.shape[0]
    return pl.pallas_call(
        functools.partial(_wide_body, relu, None),
        in_specs=[pl.BlockSpec((M, K), lambda: (0, 0)),
                  pl.BlockSpec((O, K), lambda: (0, 0)),
                  pl.BlockSpec((1, O), lambda: (0, 0))],
        out_specs=pl.BlockSpec((M, O), lambda: (0, 0)),
        out_shape=jax.ShapeDtypeStruct((M, O), jnp.float32),
    )(x, w, b.reshape(1, O))


def _lat_body(f1_ref, f2_ref, f3_ref, w_ref, b_ref, out_ref):
    w = w_ref[...]
    pre = (f1_ref[...] * w[:, 0:1] + f2_ref[...] * w[:, 1:2]
           + f3_ref[...] * w[:, 2:3] + b_ref[...])
    m = jnp.mean(pre)
    v = jnp.mean(pre * pre) - m * m
    out_ref[...] = jnp.maximum((pre - m) * jax.lax.rsqrt(v + _EPS), 0.0)


def _lat_fuse(f1, f2, f3, w, b):
    Bz, C = f1.shape
    full = lambda: (0, 0)
    spec = pl.BlockSpec((Bz, C), full)
    return pl.pallas_call(
        _lat_body,
        in_specs=[spec, spec, spec, pl.BlockSpec((1, 3), full),
                  pl.BlockSpec((1, 1), full)],
        out_specs=spec,
        out_shape=jax.ShapeDtypeStruct((Bz, C), jnp.float32),
    )(f1, f2, f3, w, b.reshape(1, 1))


def _mkstats(s, ss, m_rows):
    mean = s / m_rows
    var = jnp.maximum(ss / m_rows - mean * mean, 0.0)
    return mean, jax.lax.rsqrt(var + _EPS)


def _tower(x, p, s):
    Bz, N, _ = x.shape
    M = Bz * N
    xr = x.reshape(M, 3)
    y1, s1, ss1 = _fused(xr, p['cl%d_w1' % s], p['cl%d_b1' % s])
    st1 = _mkstats(s1, ss1, M)
    y2, s2, ss2 = _fused(y1, p['cl%d_w2' % s], p['cl%d_b2' % s], stats=st1)
    st2 = _mkstats(s2, ss2, M)
    y3, s3, ss3, mxr3 = _fused(y2, p['cl%d_w3' % s], p['cl%d_b3' % s],
                               stats=st2, pool=True, n_per_sample=N)
    st3 = _mkstats(s3, ss3, M)
    y4, s4, ss4, mxr4 = _fused(y3, p['cl%d_w4' % s], p['cl%d_b4' % s],
                               stats=st3, pool=True, n_per_sample=N)
    st4 = _mkstats(s4, ss4, M)
    y5, s5, ss5, mxr5 = _fused(y4, p['cl%d_w5' % s], p['cl%d_b5' % s],
                               stats=st4, pool=True, n_per_sample=N)
    st5 = _mkstats(s5, ss5, M)
    s6, ss6, mxr6 = _fused(y5, p['cl%d_w6' % s], p['cl%d_b6' % s],
                           stats=st5, pool=True, n_per_sample=N,
                           want_y=False)
    st6 = _mkstats(s6, ss6, M)
    g = _gcat([mxr6, mxr5, mxr4, mxr3], [st6, st5, st4, st3])
    return g, y3, st3


def kernel(x1, x2, x3, params):
    p = params
    Bz, N0, _ = x1.shape
    M0 = Bz * N0

    g1, y3_s0, st3_s0 = _tower(x1, p, 0)
    g2, _, _ = _tower(x2, p, 1)
    g3, _, _ = _tower(x3, p, 2)

    lat = _lat_fuse(g1, g2, g3, p['lf_w'], p['lf_b'])

    w0 = p['sg_w0']
    latproj = _dsmall(lat, w0[:, 128:], p['sg_b0'])
    z0, t0, tt0 = _fused(y3_s0, w0[:, :128], jnp.zeros((1088,), jnp.float32),
                         stats=st3_s0, latb=latproj, n_per_sample=N0)
    u0 = _mkstats(t0, tt0, M0)
    z1, t1, tt1 = _fused(z0, p['sg_w1'], p['sg_b1'], stats=u0)
    u1 = _mkstats(t1, tt1, M0)
    z2, t2, tt2 = _fused(z1, p['sg_w2'], p['sg_b2'], stats=u1)
    u2 = _mkstats(t2, tt2, M0)
    z3, t3, tt3 = _fused(z2, p['sg_w3'], p['sg_b3'], stats=u2)
    u3 = _mkstats(t3, tt3, M0)
    (z4,) = _fused(z3, p['sg_w4'], p['sg_b4'], stats=u3, want_stats=False,
                   softmax=True)
    seg = z4.reshape(Bz, N0, -1)

    x_1 = _dsmall(lat, p['d_w_fc1'], p['d_b_fc1'], relu=True)
    x_2 = _dsmall(x_1, p['d_w_fc2'], p['d_b_fc2'], relu=True)
    x_3 = _dsmall(x_2, p['d_w_fc3'], p['d_b_fc3'], relu=True)
    pc1_xyz = _dsmall(x_3, p['d_w_fc31'], p['d_b_fc31']).reshape(Bz, 64, 3)

    f21 = _wide(x_2, p['d_w_fc21'], p['d_b_fc21'])
    f21r = f21.reshape(Bz, 128, 64).transpose(0, 2, 1).reshape(Bz * 64, 128)
    pc2 = _dsmall(f21r, p['d_w_c21'], p['d_b_c21'])
    pc2_xyz = (pc1_xyz[:, :, None, :]
               + pc2.reshape(Bz, 64, 2, 3)).reshape(Bz, 128, 3)

    f11 = _wide(x_1, p['d_w_fc11'], p['d_b_fc11'])
    f11r = f11.reshape(Bz, 512, 128).transpose(0, 2, 1).reshape(Bz * 128, 512)
    h = _dsmall(f11r, p['d_w_c11'], p['d_b_c11'], relu=True)
    h = _dsmall(h, p['d_w_c12'], p['d_b_c12'], relu=True)
    pc3 = _dsmall(h, p['d_w_c13'], p['d_b_c13'])
    pc3_xyz = (pc2_xyz[:, :, None, :]
               + pc3.reshape(Bz, 128, 4, 3)).reshape(Bz, 512, 3)

    return (pc1_xyz, pc2_xyz, pc3_xyz, seg)


# speedup vs baseline: 1.0247x; 1.0247x over previous
import functools

import jax
import jax.numpy as jnp
from jax.experimental import pallas as pl

_EPS = 1e-5
_BM = 3072


def _fused_body(has_stats, has_latb, want_stats, want_y, pool, spb,
                softmax, *refs):
    it = iter(refs)
    x_ref = next(it)
    w_ref = next(it)
    b_ref = next(it)
    mu_ref = next(it) if has_stats else None
    isig_ref = next(it) if has_stats else None
    latb_ref = next(it) if has_latb else None
    y_ref = next(it) if want_y else None
    sum_ref = next(it) if want_stats else None
    ssq_ref = next(it) if want_stats else None
    ymax_ref = next(it) if pool else None

    i = pl.program_id(0)
    a = x_ref[...]
    if has_stats:
        a = jnp.maximum((a - mu_ref[...]) * isig_ref[...], 0.0)
    y = jax.lax.dot_general(a, w_ref[...], (((1,), (1,)), ((), ())),
                            preferred_element_type=jnp.float32)
    y = y + b_ref[...]
    if has_latb:
        bm, o = y.shape
        y = (y.reshape(spb, bm // spb, o) + latb_ref[...]).reshape(bm, o)
    if softmax:
        m = jnp.max(y, axis=1, keepdims=True)
        y = y - (m + jnp.log(jnp.sum(jnp.exp(y - m), axis=1, keepdims=True)))
    if want_y:
        y_ref[...] = y
    if want_stats:
        s = jnp.sum(y, axis=0, keepdims=True)
        ss = jnp.sum(y * y, axis=0, keepdims=True)

        @pl.when(i == 0)
        def _():
            sum_ref[...] = s
            ssq_ref[...] = ss

        @pl.when(i > 0)
        def _():
            sum_ref[...] += s
            ssq_ref[...] += ss
    if pool:
        bm, o = y.shape
        ymax_ref[...] = jnp.max(y.reshape(spb, bm // spb, o), axis=1)[:, None]


def _fused(x, w, b, stats=None, latb=None, pool=False, n_per_sample=None,
           bm=_BM, want_stats=True, want_y=True, softmax=False):
    M, K = x.shape
    O = w.shape[0]
    nt = M // bm
    spb = (bm // n_per_sample) if n_per_sample else 1
    nsamp = (M // n_per_sample) if n_per_sample else 1

    in_specs = [
        pl.BlockSpec((bm, K), lambda i: (i, 0)),
        pl.BlockSpec((O, K), lambda i: (0, 0)),
        pl.BlockSpec((1, O), lambda i: (0, 0)),
    ]
    args = [x, w, b.reshape(1, O)]
    if stats is not None:
        in_specs += [pl.BlockSpec((1, K), lambda i: (0, 0)),
                     pl.BlockSpec((1, K), lambda i: (0, 0))]
        args += [stats[0], stats[1]]
    if latb is not None:
        in_specs += [pl.BlockSpec((spb, 1, O), lambda i: (i, 0, 0))]
        args += [latb.reshape(nsamp, 1, O)]

    out_shape, out_specs = [], []
    if want_y:
        out_shape += [jax.ShapeDtypeStruct((M, O), jnp.float32)]
        out_specs += [pl.BlockSpec((bm, O), lambda i: (i, 0))]
    if want_stats:
        out_shape += [jax.ShapeDtypeStruct((1, O), jnp.float32)] * 2
        out_specs += [pl.BlockSpec((1, O), lambda i: (0, 0))] * 2
    if pool:
        out_shape += [jax.ShapeDtypeStruct((nsamp, 1, O), jnp.float32)]
        out_specs += [pl.BlockSpec((spb, 1, O), lambda i: (i, 0, 0))]

    body = functools.partial(_fused_body, stats is not None,
                             latb is not None, want_stats, want_y, pool, spb,
                             softmax)
    outs = pl.pallas_call(
        body, grid=(nt,), in_specs=in_specs, out_specs=out_specs,
        out_shape=out_shape)(*args)
    outs = list(outs)
    if pool:
        outs[-1] = outs[-1].reshape(nsamp, O)
    return outs


def _chain2_body(spb, x_ref, w0_ref, latb_ref, mu3_ref, is3_ref, mu0_ref,
                 is0_ref, w1_ref, b1_ref, y_ref, sum_ref, ssq_ref):
    i = pl.program_id(0)
    a = jnp.maximum((x_ref[...] - mu3_ref[...]) * is3_ref[...], 0.0)
    z0 = jax.lax.dot_general(a, w0_ref[...], (((1,), (1,)), ((), ())),
                             preferred_element_type=jnp.float32)
    bm, o0 = z0.shape
    z0 = (z0.reshape(spb, bm // spb, o0) + latb_ref[...]).reshape(bm, o0)
    a1 = jnp.maximum((z0 - mu0_ref[...]) * is0_ref[...], 0.0)
    y = jax.lax.dot_general(a1, w1_ref[...], (((1,), (1,)), ((), ())),
                            preferred_element_type=jnp.float32) + b1_ref[...]
    y_ref[...] = y
    s = jnp.sum(y, axis=0, keepdims=True)
    ss = jnp.sum(y * y, axis=0, keepdims=True)

    @pl.when(i == 0)
    def _():
        sum_ref[...] = s
        ssq_ref[...] = ss

    @pl.when(i > 0)
    def _():
        sum_ref[...] += s
        ssq_ref[...] += ss


def _chain2(x, w0, latb, st_in, st0, w1, b1, n_per_sample, bm=_BM):
    M, K = x.shape
    O0 = w0.shape[0]
    O = w1.shape[0]
    nt = M // bm
    spb = bm // n_per_sample
    nsamp = M // n_per_sample
    outs = pl.pallas_call(
        functools.partial(_chain2_body, spb),
        grid=(nt,),
        in_specs=[
            pl.BlockSpec((bm, K), lambda i: (i, 0)),
            pl.BlockSpec((O0, K), lambda i: (0, 0)),
            pl.BlockSpec((spb, 1, O0), lambda i: (i, 0, 0)),
            pl.BlockSpec((1, K), lambda i: (0, 0)),
            pl.BlockSpec((1, K), lambda i: (0, 0)),
            pl.BlockSpec((1, O0), lambda i: (0, 0)),
            pl.BlockSpec((1, O0), lambda i: (0, 0)),
            pl.BlockSpec((O, O0), lambda i: (0, 0)),
            pl.BlockSpec((1, O), lambda i: (0, 0)),
        ],
        out_specs=[pl.BlockSpec((bm, O), lambda i: (i, 0)),
                   pl.BlockSpec((1, O), lambda i: (0, 0)),
                   pl.BlockSpec((1, O), lambda i: (0, 0))],
        out_shape=[jax.ShapeDtypeStruct((M, O), jnp.float32),
                   jax.ShapeDtypeStruct((1, O), jnp.float32),
                   jax.ShapeDtypeStruct((1, O), jnp.float32)],
    )(x, w0, latb.reshape(nsamp, 1, O0), st_in[0], st_in[1], st0[0], st0[1],
      w1, b1.reshape(1, O))
    return outs


def _gcat_body(*refs):
    (m6, mu6, is6, m5, mu5, is5, m4, mu4, is4, m3, mu3, is3, out) = refs
    out[:, 0:1024] = jnp.maximum((m6[...] - mu6[...]) * is6[...], 0.0)
    out[:, 1024:1536] = jnp.maximum((m5[...] - mu5[...]) * is5[...], 0.0)
    out[:, 1536:1792] = jnp.maximum((m4[...] - mu4[...]) * is4[...], 0.0)
    out[:, 1792:1920] = jnp.maximum((m3[...] - mu3[...]) * is3[...], 0.0)


def _gcat(maxes, stats):
    nsamp = maxes[0].shape[0]
    args, in_specs = [], []
    for m, st in zip(maxes, stats):
        args += [m, st[0], st[1]]
        in_specs += [pl.BlockSpec(m.shape, lambda: (0, 0)),
                     pl.BlockSpec(st[0].shape, lambda: (0, 0)),
                     pl.BlockSpec(st[1].shape, lambda: (0, 0))]
    return pl.pallas_call(
        _gcat_body,
        in_specs=in_specs,
        out_specs=pl.BlockSpec((nsamp, 1920), lambda: (0, 0)),
        out_shape=jax.ShapeDtypeStruct((nsamp, 1920), jnp.float32),
    )(*args)


def _wide_body(relu, precision, x_ref, w_ref, b_ref, y_ref):
    y = jax.lax.dot_general(x_ref[...], w_ref[...], (((1,), (1,)), ((), ())),
                            preferred_element_type=jnp.float32,
                            precision=precision) + b_ref[...]
    if relu:
        y = jnp.maximum(y, 0.0)
    y_ref[...] = y


def _wide(x, w, b, bo=2048, relu=True):
    M, K = x.shape
    O = w.shape[0]
    nt = O // bo
    return pl.pallas_call(
        functools.partial(_wide_body, relu, None),
        grid=(nt,),
        in_specs=[pl.BlockSpec((M, K), lambda i: (0, 0)),
                  pl.BlockSpec((bo, K), lambda i: (i, 0)),
                  pl.BlockSpec((1, bo), lambda i: (0, i))],
        out_specs=pl.BlockSpec((M, bo), lambda i: (0, i)),
        out_shape=jax.ShapeDtypeStruct((M, O), jnp.float32),
    )(x, w, b.reshape(1, O))


def _dsmall(x, w, b, relu=False):
    M, K = x.shape
    O = w.shape[0]
    return pl.pallas_call(
        functools.partial(_wide_body, relu, None),
        in_specs=[pl.BlockSpec((M, K), lambda: (0, 0)),
                  pl.BlockSpec((O, K), lambda: (0, 0)),
                  pl.BlockSpec((1, O), lambda: (0, 0))],
        out_specs=pl.BlockSpec((M, O), lambda: (0, 0)),
        out_shape=jax.ShapeDtypeStruct((M, O), jnp.float32),
    )(x, w, b.reshape(1, O))


def _lat_body(f1_ref, f2_ref, f3_ref, w_ref, b_ref, out_ref):
    w = w_ref[...]
    pre = (f1_ref[...] * w[:, 0:1] + f2_ref[...] * w[:, 1:2]
           + f3_ref[...] * w[:, 2:3] + b_ref[...])
    m = jnp.mean(pre)
    v = jnp.mean(pre * pre) - m * m
    out_ref[...] = jnp.maximum((pre - m) * jax.lax.rsqrt(v + _EPS), 0.0)


def _lat_fuse(f1, f2, f3, w, b):
    Bz, C = f1.shape
    full = lambda: (0, 0)
    spec = pl.BlockSpec((Bz, C), full)
    return pl.pallas_call(
        _lat_body,
        in_specs=[spec, spec, spec, pl.BlockSpec((1, 3), full),
                  pl.BlockSpec((1, 1), full)],
        out_specs=spec,
        out_shape=jax.ShapeDtypeStruct((Bz, C), jnp.float32),
    )(f1, f2, f3, w, b.reshape(1, 1))


def _mkstats(s, ss, m_rows):
    mean = s / m_rows
    var = jnp.maximum(ss / m_rows - mean * mean, 0.0)
    return mean, jax.lax.rsqrt(var + _EPS)


def _tower(x, p, s):
    Bz, N, _ = x.shape
    M = Bz * N
    xr = x.reshape(M, 3)
    y1, s1, ss1 = _fused(xr, p['cl%d_w1' % s], p['cl%d_b1' % s])
    st1 = _mkstats(s1, ss1, M)
    y2, s2, ss2 = _fused(y1, p['cl%d_w2' % s], p['cl%d_b2' % s], stats=st1)
    st2 = _mkstats(s2, ss2, M)
    y3, s3, ss3, mxr3 = _fused(y2, p['cl%d_w3' % s], p['cl%d_b3' % s],
                               stats=st2, pool=True, n_per_sample=N)
    st3 = _mkstats(s3, ss3, M)
    y4, s4, ss4, mxr4 = _fused(y3, p['cl%d_w4' % s], p['cl%d_b4' % s],
                               stats=st3, pool=True, n_per_sample=N)
    st4 = _mkstats(s4, ss4, M)
    y5, s5, ss5, mxr5 = _fused(y4, p['cl%d_w5' % s], p['cl%d_b5' % s],
                               stats=st4, pool=True, n_per_sample=N)
    st5 = _mkstats(s5, ss5, M)
    s6, ss6, mxr6 = _fused(y5, p['cl%d_w6' % s], p['cl%d_b6' % s],
                           stats=st5, pool=True, n_per_sample=N,
                           want_y=False)
    st6 = _mkstats(s6, ss6, M)
    g = _gcat([mxr6, mxr5, mxr4, mxr3], [st6, st5, st4, st3])
    return g, y3, st3


def kernel(x1, x2, x3, params):
    p = params
    Bz, N0, _ = x1.shape
    M0 = Bz * N0

    g1, y3_s0, st3_s0 = _tower(x1, p, 0)
    g2, _, _ = _tower(x2, p, 1)
    g3, _, _ = _tower(x3, p, 2)

    lat = _lat_fuse(g1, g2, g3, p['lf_w'], p['lf_b'])

    w0 = p['sg_w0']
    latproj = _dsmall(lat, w0[:, 128:], p['sg_b0'])
    t0, tt0 = _fused(y3_s0, w0[:, :128], jnp.zeros((1088,), jnp.float32),
                     stats=st3_s0, latb=latproj, n_per_sample=N0,
                     want_y=False)
    u0 = _mkstats(t0, tt0, M0)
    z1, t1, tt1 = _chain2(y3_s0, w0[:, :128], latproj, st3_s0, u0,
                          p['sg_w1'], p['sg_b1'], N0)
    u1 = _mkstats(t1, tt1, M0)
    z2, t2, tt2 = _fused(z1, p['sg_w2'], p['sg_b2'], stats=u1)
    u2 = _mkstats(t2, tt2, M0)
    z3, t3, tt3 = _fused(z2, p['sg_w3'], p['sg_b3'], stats=u2)
    u3 = _mkstats(t3, tt3, M0)
    (z4,) = _fused(z3, p['sg_w4'], p['sg_b4'], stats=u3, want_stats=False,
                   softmax=True)
    seg = z4.reshape(Bz, N0, -1)

    x_1 = _dsmall(lat, p['d_w_fc1'], p['d_b_fc1'], relu=True)
    x_2 = _dsmall(x_1, p['d_w_fc2'], p['d_b_fc2'], relu=True)
    x_3 = _dsmall(x_2, p['d_w_fc3'], p['d_b_fc3'], relu=True)
    pc1_xyz = _dsmall(x_3, p['d_w_fc31'], p['d_b_fc31']).reshape(Bz, 64, 3)

    f21 = _wide(x_2, p['d_w_fc21'], p['d_b_fc21'])
    f21r = f21.reshape(Bz, 128, 64).transpose(0, 2, 1).reshape(Bz * 64, 128)
    pc2 = _dsmall(f21r, p['d_w_c21'], p['d_b_c21'])
    pc2_xyz = (pc1_xyz[:, :, None, :]
               + pc2.reshape(Bz, 64, 2, 3)).reshape(Bz, 128, 3)

    f11 = _wide(x_1, p['d_w_fc11'], p['d_b_fc11'])
    f11r = f11.reshape(Bz, 512, 128).transpose(0, 2, 1).reshape(Bz * 128, 512)
    h = _dsmall(f11r, p['d_w_c11'], p['d_b_c11'], relu=True)
    h = _dsmall(h, p['d_w_c12'], p['d_b_c12'], relu=True)
    pc3 = _dsmall(h, p['d_w_c13'], p['d_b_c13'])
    pc3_xyz = (pc2_xyz[:, :, None, :]
               + pc3.reshape(Bz, 128, 4, 3)).reshape(Bz, 512, 3)

    return (pc1_xyz, pc2_xyz, pc3_xyz, seg)


# speedup vs baseline: 1.0412x; 1.0161x over previous
import functools

import jax
import jax.numpy as jnp
from jax.experimental import pallas as pl

_EPS = 1e-5
_BM = 3072


def _fused_body(has_stats, has_latb, want_stats, want_y, pool, spb,
                softmax, *refs):
    it = iter(refs)
    x_ref = next(it)
    w_ref = next(it)
    b_ref = next(it)
    mu_ref = next(it) if has_stats else None
    isig_ref = next(it) if has_stats else None
    latb_ref = next(it) if has_latb else None
    y_ref = next(it) if want_y else None
    sum_ref = next(it) if want_stats else None
    ssq_ref = next(it) if want_stats else None
    ymax_ref = next(it) if pool else None

    i = pl.program_id(0)
    a = x_ref[...]
    if has_stats:
        a = jnp.maximum((a - mu_ref[...]) * isig_ref[...], 0.0)
    y = jax.lax.dot_general(a, w_ref[...], (((1,), (1,)), ((), ())),
                            preferred_element_type=jnp.float32)
    y = y + b_ref[...]
    if has_latb:
        bm, o = y.shape
        y = (y.reshape(spb, bm // spb, o) + latb_ref[...]).reshape(bm, o)
    if softmax:
        m = jnp.max(y, axis=1, keepdims=True)
        y = y - (m + jnp.log(jnp.sum(jnp.exp(y - m), axis=1, keepdims=True)))
    if want_y:
        y_ref[...] = y
    if want_stats:
        s = jnp.sum(y, axis=0, keepdims=True)
        ss = jnp.sum(y * y, axis=0, keepdims=True)

        @pl.when(i == 0)
        def _():
            sum_ref[...] = s
            ssq_ref[...] = ss

        @pl.when(i > 0)
        def _():
            sum_ref[...] += s
            ssq_ref[...] += ss
    if pool:
        bm, o = y.shape
        ymax_ref[...] = jnp.max(y.reshape(spb, bm // spb, o), axis=1)[:, None]


def _fused(x, w, b, stats=None, latb=None, pool=False, n_per_sample=None,
           bm=_BM, want_stats=True, want_y=True, softmax=False):
    M, K = x.shape
    O = w.shape[0]
    nt = M // bm
    spb = (bm // n_per_sample) if n_per_sample else 1
    nsamp = (M // n_per_sample) if n_per_sample else 1

    in_specs = [
        pl.BlockSpec((bm, K), lambda i: (i, 0)),
        pl.BlockSpec((O, K), lambda i: (0, 0)),
        pl.BlockSpec((1, O), lambda i: (0, 0)),
    ]
    args = [x, w, b.reshape(1, O)]
    if stats is not None:
        in_specs += [pl.BlockSpec((1, K), lambda i: (0, 0)),
                     pl.BlockSpec((1, K), lambda i: (0, 0))]
        args += [stats[0], stats[1]]
    if latb is not None:
        in_specs += [pl.BlockSpec((spb, 1, O), lambda i: (i, 0, 0))]
        args += [latb.reshape(nsamp, 1, O)]

    out_shape, out_specs = [], []
    if want_y:
        out_shape += [jax.ShapeDtypeStruct((M, O), jnp.float32)]
        out_specs += [pl.BlockSpec((bm, O), lambda i: (i, 0))]
    if want_stats:
        out_shape += [jax.ShapeDtypeStruct((1, O), jnp.float32)] * 2
        out_specs += [pl.BlockSpec((1, O), lambda i: (0, 0))] * 2
    if pool:
        out_shape += [jax.ShapeDtypeStruct((nsamp, 1, O), jnp.float32)]
        out_specs += [pl.BlockSpec((spb, 1, O), lambda i: (i, 0, 0))]

    body = functools.partial(_fused_body, stats is not None,
                             latb is not None, want_stats, want_y, pool, spb,
                             softmax)
    outs = pl.pallas_call(
        body, grid=(nt,), in_specs=in_specs, out_specs=out_specs,
        out_shape=out_shape)(*args)
    outs = list(outs)
    if pool:
        outs[-1] = outs[-1].reshape(nsamp, O)
    return outs


def _chain2_body(spb, x_ref, w0_ref, latb_ref, mu3_ref, is3_ref, mu0_ref,
                 is0_ref, w1_ref, b1_ref, y_ref, sum_ref, ssq_ref):
    i = pl.program_id(0)
    a = jnp.maximum((x_ref[...] - mu3_ref[...]) * is3_ref[...], 0.0)
    z0 = jax.lax.dot_general(a, w0_ref[...], (((1,), (1,)), ((), ())),
                             preferred_element_type=jnp.float32)
    bm, o0 = z0.shape
    z0 = (z0.reshape(spb, bm // spb, o0) + latb_ref[...]).reshape(bm, o0)
    a1 = jnp.maximum((z0 - mu0_ref[...]) * is0_ref[...], 0.0)
    y = jax.lax.dot_general(a1, w1_ref[...], (((1,), (1,)), ((), ())),
                            preferred_element_type=jnp.float32) + b1_ref[...]
    y_ref[...] = y
    s = jnp.sum(y, axis=0, keepdims=True)
    ss = jnp.sum(y * y, axis=0, keepdims=True)

    @pl.when(i == 0)
    def _():
        sum_ref[...] = s
        ssq_ref[...] = ss

    @pl.when(i > 0)
    def _():
        sum_ref[...] += s
        ssq_ref[...] += ss


def _chain2(x, w0, latb, st_in, st0, w1, b1, n_per_sample, bm=_BM):
    M, K = x.shape
    O0 = w0.shape[0]
    O = w1.shape[0]
    nt = M // bm
    spb = bm // n_per_sample
    nsamp = M // n_per_sample
    outs = pl.pallas_call(
        functools.partial(_chain2_body, spb),
        grid=(nt,),
        in_specs=[
            pl.BlockSpec((bm, K), lambda i: (i, 0)),
            pl.BlockSpec((O0, K), lambda i: (0, 0)),
            pl.BlockSpec((spb, 1, O0), lambda i: (i, 0, 0)),
            pl.BlockSpec((1, K), lambda i: (0, 0)),
            pl.BlockSpec((1, K), lambda i: (0, 0)),
            pl.BlockSpec((1, O0), lambda i: (0, 0)),
            pl.BlockSpec((1, O0), lambda i: (0, 0)),
            pl.BlockSpec((O, O0), lambda i: (0, 0)),
            pl.BlockSpec((1, O), lambda i: (0, 0)),
        ],
        out_specs=[pl.BlockSpec((bm, O), lambda i: (i, 0)),
                   pl.BlockSpec((1, O), lambda i: (0, 0)),
                   pl.BlockSpec((1, O), lambda i: (0, 0))],
        out_shape=[jax.ShapeDtypeStruct((M, O), jnp.float32),
                   jax.ShapeDtypeStruct((1, O), jnp.float32),
                   jax.ShapeDtypeStruct((1, O), jnp.float32)],
    )(x, w0, latb.reshape(nsamp, 1, O0), st_in[0], st_in[1], st0[0], st0[1],
      w1, b1.reshape(1, O))
    return outs


def _gcat_body(*refs):
    (m6, mu6, is6, m5, mu5, is5, m4, mu4, is4, m3, mu3, is3, out) = refs
    out[:, 0:1024] = jnp.maximum((m6[...] - mu6[...]) * is6[...], 0.0)
    out[:, 1024:1536] = jnp.maximum((m5[...] - mu5[...]) * is5[...], 0.0)
    out[:, 1536:1792] = jnp.maximum((m4[...] - mu4[...]) * is4[...], 0.0)
    out[:, 1792:1920] = jnp.maximum((m3[...] - mu3[...]) * is3[...], 0.0)


def _gcat(maxes, stats):
    nsamp = maxes[0].shape[0]
    args, in_specs = [], []
    for m, st in zip(maxes, stats):
        args += [m, st[0], st[1]]
        in_specs += [pl.BlockSpec(m.shape, lambda: (0, 0)),
                     pl.BlockSpec(st[0].shape, lambda: (0, 0)),
                     pl.BlockSpec(st[1].shape, lambda: (0, 0))]
    return pl.pallas_call(
        _gcat_body,
        in_specs=in_specs,
        out_specs=pl.BlockSpec((nsamp, 1920), lambda: (0, 0)),
        out_shape=jax.ShapeDtypeStruct((nsamp, 1920), jnp.float32),
    )(*args)


def _dot(a, w):
    return jax.lax.dot_general(a, w, (((1,), (1,)), ((), ())),
                               preferred_element_type=jnp.float32)


def _wide_body(relu, precision, x_ref, w_ref, b_ref, y_ref):
    y = jax.lax.dot_general(x_ref[...], w_ref[...], (((1,), (1,)), ((), ())),
                            preferred_element_type=jnp.float32,
                            precision=precision) + b_ref[...]
    if relu:
        y = jnp.maximum(y, 0.0)
    y_ref[...] = y


def _dec_head_body(lat_ref, w1, b1, w2, b2, w3, b3, w31, b31,
                   x1_ref, x2_ref, pc1_ref):
    x1 = jnp.maximum(_dot(lat_ref[...], w1[...]) + b1[...], 0.0)
    x1_ref[...] = x1
    x2 = jnp.maximum(_dot(x1, w2[...]) + b2[...], 0.0)
    x2_ref[...] = x2
    x3 = jnp.maximum(_dot(x2, w3[...]) + b3[...], 0.0)
    pc1_ref[...] = _dot(x3, w31[...]) + b31[...]


def _dec_head(lat, p):
    Bz = lat.shape[0]
    names = ['fc1', 'fc2', 'fc3', 'fc31']
    args, in_specs = [lat], [pl.BlockSpec(lat.shape, lambda: (0, 0))]
    for n in names:
        w, b = p['d_w_%s' % n], p['d_b_%s' % n]
        args += [w, b.reshape(1, -1)]
        in_specs += [pl.BlockSpec(w.shape, lambda: (0, 0)),
                     pl.BlockSpec((1, b.shape[0]), lambda: (0, 0))]
    return pl.pallas_call(
        _dec_head_body,
        in_specs=in_specs,
        out_specs=[pl.BlockSpec((Bz, 1024), lambda: (0, 0)),
                   pl.BlockSpec((Bz, 512), lambda: (0, 0)),
                   pl.BlockSpec((Bz, 192), lambda: (0, 0))],
        out_shape=[jax.ShapeDtypeStruct((Bz, 1024), jnp.float32),
                   jax.ShapeDtypeStruct((Bz, 512), jnp.float32),
                   jax.ShapeDtypeStruct((Bz, 192), jnp.float32)],
    )(*args)


def _c3_body(f_ref, w11, b11, w12, b12, w13, b13, out_ref):
    h = jnp.maximum(_dot(f_ref[...], w11[...]) + b11[...], 0.0)
    h = jnp.maximum(_dot(h, w12[...]) + b12[...], 0.0)
    out_ref[...] = _dot(h, w13[...]) + b13[...]


def _c3(f, p):
    M = f.shape[0]
    args, in_specs = [f], [pl.BlockSpec(f.shape, lambda: (0, 0))]
    for n in ['c11', 'c12', 'c13']:
        w, b = p['d_w_%s' % n], p['d_b_%s' % n]
        args += [w, b.reshape(1, -1)]
        in_specs += [pl.BlockSpec(w.shape, lambda: (0, 0)),
                     pl.BlockSpec((1, b.shape[0]), lambda: (0, 0))]
    return pl.pallas_call(
        _c3_body,
        in_specs=in_specs,
        out_specs=pl.BlockSpec((M, 12), lambda: (0, 0)),
        out_shape=jax.ShapeDtypeStruct((M, 12), jnp.float32),
    )(*args)


def _wide(x, w, b, bo=2048, relu=True):
    M, K = x.shape
    O = w.shape[0]
    nt = O // bo
    return pl.pallas_call(
        functools.partial(_wide_body, relu, None),
        grid=(nt,),
        in_specs=[pl.BlockSpec((M, K), lambda i: (0, 0)),
                  pl.BlockSpec((bo, K), lambda i: (i, 0)),
                  pl.BlockSpec((1, bo), lambda i: (0, i))],
        out_specs=pl.BlockSpec((M, bo), lambda i: (0, i)),
        out_shape=jax.ShapeDtypeStruct((M, O), jnp.float32),
    )(x, w, b.reshape(1, O))


def _dsmall(x, w, b, relu=False):
    M, K = x.shape
    O = w.shape[0]
    return pl.pallas_call(
        functools.partial(_wide_body, relu, None),
        in_specs=[pl.BlockSpec((M, K), lambda: (0, 0)),
                  pl.BlockSpec((O, K), lambda: (0, 0)),
                  pl.BlockSpec((1, O), lambda: (0, 0))],
        out_specs=pl.BlockSpec((M, O), lambda: (0, 0)),
        out_shape=jax.ShapeDtypeStruct((M, O), jnp.float32),
    )(x, w, b.reshape(1, O))


def _lat_body(f1_ref, f2_ref, f3_ref, w_ref, b_ref, out_ref):
    w = w_ref[...]
    pre = (f1_ref[...] * w[:, 0:1] + f2_ref[...] * w[:, 1:2]
           + f3_ref[...] * w[:, 2:3] + b_ref[...])
    m = jnp.mean(pre)
    v = jnp.mean(pre * pre) - m * m
    out_ref[...] = jnp.maximum((pre - m) * jax.lax.rsqrt(v + _EPS), 0.0)


def _lat_fuse(f1, f2, f3, w, b):
    Bz, C = f1.shape
    full = lambda: (0, 0)
    spec = pl.BlockSpec((Bz, C), full)
    return pl.pallas_call(
        _lat_body,
        in_specs=[spec, spec, spec, pl.BlockSpec((1, 3), full),
                  pl.BlockSpec((1, 1), full)],
        out_specs=spec,
        out_shape=jax.ShapeDtypeStruct((Bz, C), jnp.float32),
    )(f1, f2, f3, w, b.reshape(1, 1))


def _mkstats(s, ss, m_rows):
    mean = s / m_rows
    var = jnp.maximum(ss / m_rows - mean * mean, 0.0)
    return mean, jax.lax.rsqrt(var + _EPS)


def _tower(x, p, s):
    Bz, N, _ = x.shape
    M = Bz * N
    xr = x.reshape(M, 3)
    y1, s1, ss1 = _fused(xr, p['cl%d_w1' % s], p['cl%d_b1' % s])
    st1 = _mkstats(s1, ss1, M)
    y2, s2, ss2 = _fused(y1, p['cl%d_w2' % s], p['cl%d_b2' % s], stats=st1)
    st2 = _mkstats(s2, ss2, M)
    y3, s3, ss3, mxr3 = _fused(y2, p['cl%d_w3' % s], p['cl%d_b3' % s],
                               stats=st2, pool=True, n_per_sample=N)
    st3 = _mkstats(s3, ss3, M)
    y4, s4, ss4, mxr4 = _fused(y3, p['cl%d_w4' % s], p['cl%d_b4' % s],
                               stats=st3, pool=True, n_per_sample=N)
    st4 = _mkstats(s4, ss4, M)
    y5, s5, ss5, mxr5 = _fused(y4, p['cl%d_w5' % s], p['cl%d_b5' % s],
                               stats=st4, pool=True, n_per_sample=N)
    st5 = _mkstats(s5, ss5, M)
    s6, ss6, mxr6 = _fused(y5, p['cl%d_w6' % s], p['cl%d_b6' % s],
                           stats=st5, pool=True, n_per_sample=N,
                           want_y=False)
    st6 = _mkstats(s6, ss6, M)
    g = _gcat([mxr6, mxr5, mxr4, mxr3], [st6, st5, st4, st3])
    return g, y3, st3


def kernel(x1, x2, x3, params):
    p = params
    Bz, N0, _ = x1.shape
    M0 = Bz * N0

    g1, y3_s0, st3_s0 = _tower(x1, p, 0)
    g2, _, _ = _tower(x2, p, 1)
    g3, _, _ = _tower(x3, p, 2)

    lat = _lat_fuse(g1, g2, g3, p['lf_w'], p['lf_b'])

    w0 = p['sg_w0']
    latproj = _dsmall(lat, w0[:, 128:], p['sg_b0'])
    t0, tt0 = _fused(y3_s0, w0[:, :128], jnp.zeros((1088,), jnp.float32),
                     stats=st3_s0, latb=latproj, n_per_sample=N0,
                     want_y=False)
    u0 = _mkstats(t0, tt0, M0)
    z1, t1, tt1 = _chain2(y3_s0, w0[:, :128], latproj, st3_s0, u0,
                          p['sg_w1'], p['sg_b1'], N0)
    u1 = _mkstats(t1, tt1, M0)
    z2, t2, tt2 = _fused(z1, p['sg_w2'], p['sg_b2'], stats=u1)
    u2 = _mkstats(t2, tt2, M0)
    z3, t3, tt3 = _fused(z2, p['sg_w3'], p['sg_b3'], stats=u2)
    u3 = _mkstats(t3, tt3, M0)
    (z4,) = _fused(z3, p['sg_w4'], p['sg_b4'], stats=u3, want_stats=False,
                   softmax=True)
    seg = z4.reshape(Bz, N0, -1)

    x_1, x_2, pc1 = _dec_head(lat, p)
    pc1_xyz = pc1.reshape(Bz, 64, 3)

    f21 = _wide(x_2, p['d_w_fc21'], p['d_b_fc21'])
    f21r = f21.reshape(Bz, 128, 64).transpose(0, 2, 1).reshape(Bz * 64, 128)
    pc2 = _dsmall(f21r, p['d_w_c21'], p['d_b_c21'])
    pc2_xyz = (pc1_xyz[:, :, None, :]
               + pc2.reshape(Bz, 64, 2, 3)).reshape(Bz, 128, 3)

    f11 = _wide(x_1, p['d_w_fc11'], p['d_b_fc11'], bo=4096)
    f11r = f11.reshape(Bz, 512, 128).transpose(0, 2, 1).reshape(Bz * 128, 512)
    pc3 = _c3(f11r, p)
    pc3_xyz = (pc2_xyz[:, :, None, :]
               + pc3.reshape(Bz, 128, 4, 3)).reshape(Bz, 512, 3)

    return (pc1_xyz, pc2_xyz, pc3_xyz, seg)
